# baseline (speedup 1.0000x reference)
import math

import jax
import jax.numpy as jnp
from jax.experimental import pallas as pl

HIDDEN = 256
IMAGE_SIZE = 1008
GRID = 72
B = 32
NPTS = 65
CB = 128
TWO_PI = 2.0 * math.pi


def _body(px_ref, py_ref, lab_ref, pos_ref, tab_ref, nap_ref, nm_ref,
          sparse_ref, dense_ref):
    i = pl.program_id(0)
    j = pl.program_id(1)

    nm = nm_ref[...].reshape(1, CB, 1, 1)
    dense_ref[...] = jnp.broadcast_to(nm, (1, CB, GRID, GRID))

    @pl.when(jnp.logical_and(i == 0, j == 0))
    def _():
        px = px_ref[...]
        py = py_ref[...]
        labels = lab_ref[...]
        inv = 1.0 / IMAGE_SIZE
        bf = jnp.bfloat16
        cx = (2.0 * (px * inv) - 1.0).astype(bf).astype(jnp.float32)
        cy = (2.0 * (py * inv) - 1.0).astype(bf).astype(jnp.float32)
        p0 = pos_ref[0:1, :].reshape(1, 1, HIDDEN // 2)
        p1 = pos_ref[1:2, :].reshape(1, 1, HIDDEN // 2)
        p0 = p0.astype(bf).astype(jnp.float32)
        p1 = p1.astype(bf).astype(jnp.float32)
        c = TWO_PI * (cx * p0 + cy * p1)
        pe = jnp.concatenate([jnp.sin(c), jnp.cos(c)], axis=-1)
        nap = nap_ref[...].reshape(1, 1, HIDDEN)
        pe = jnp.where(labels == -1, nap, pe)
        pe = pe * (labels != -10).astype(pe.dtype)
        lc = jnp.maximum(labels, 0)
        e0 = tab_ref[0:1, :].reshape(1, 1, HIDDEN)
        e1 = tab_ref[1:2, :].reshape(1, 1, HIDDEN)
        e2 = tab_ref[2:3, :].reshape(1, 1, HIDDEN)
        e3 = tab_ref[3:4, :].reshape(1, 1, HIDDEN)
        pemb = jnp.where(lc == 0, e0,
                         jnp.where(lc == 1, e1,
                                   jnp.where(lc == 2, e2, e3)))
        is_pos = (labels >= 0).astype(pe.dtype)
        sparse_ref[...] = pe + pemb * is_pos


def kernel(input_points, input_labels, positional_embedding, point_embed,
           not_a_point_embed, no_mask_embed):
    pts = input_points + 0.5
    pts = jnp.concatenate([pts, jnp.zeros((B, 1, 2), pts.dtype)], axis=1)
    px = pts[..., 0:1]
    py = pts[..., 1:2]
    labels = jnp.concatenate(
        [input_labels, -jnp.ones((B, 1), input_labels.dtype)],
        axis=1)[..., None]

    grid = (B, HIDDEN // CB)
    sparse, dense = pl.pallas_call(
        _body,
        grid=grid,
        in_specs=[
            pl.BlockSpec((B, NPTS, 1), lambda i, j: (0, 0, 0)),
            pl.BlockSpec((B, NPTS, 1), lambda i, j: (0, 0, 0)),
            pl.BlockSpec((B, NPTS, 1), lambda i, j: (0, 0, 0)),
            pl.BlockSpec((2, HIDDEN // 2), lambda i, j: (0, 0)),
            pl.BlockSpec((4, HIDDEN), lambda i, j: (0, 0)),
            pl.BlockSpec((1, HIDDEN), lambda i, j: (0, 0)),
            pl.BlockSpec((1, CB), lambda i, j: (0, j)),
        ],
        out_specs=[
            pl.BlockSpec((B, NPTS, HIDDEN), lambda i, j: (0, 0, 0)),
            pl.BlockSpec((1, CB, GRID, GRID), lambda i, j: (i, j, 0, 0)),
        ],
        out_shape=[
            jax.ShapeDtypeStruct((B, NPTS, HIDDEN), jnp.float32),
            jax.ShapeDtypeStruct((B, HIDDEN, GRID, GRID), jnp.float32),
        ],
    )(px, py, labels, positional_embedding, point_embed,
      not_a_point_embed, no_mask_embed)
    return sparse, dense
